# baseline (speedup 1.0000x reference)
import jax
import jax.numpy as jnp
from jax import lax
from jax.experimental import pallas as pl
from jax.experimental.pallas import tpu as pltpu
from jax.experimental.pallas import tpu_sc as plsc

_B, _D = 8192, 2048
_N_IN = 4
_NC, _NS = 2, 16
_NW = _NC * _NS
_ROWS_W = _B // _NW
_CHUNK = 8
_NCH = _ROWS_W // _CHUNK
_NSLOT = 6


def _mux_body(x0, x1, x2, x3, sel_hbm, out, sel_v, *bufs_and_sems):
    xs = (x0, x1, x2, x3)
    tile_bufs = bufs_and_sems[:_NSLOT]
    rsems = bufs_and_sems[_NSLOT:2 * _NSLOT]
    wsems = bufs_and_sems[2 * _NSLOT:]

    sid = lax.axis_index("s")
    wid = sid * _NC + lax.axis_index("c")
    base = wid * _ROWS_W

    pltpu.sync_copy(sel_hbm, sel_v)
    selv = sel_v[...]
    preds = [jnp.any(selv == i) for i in range(_N_IN)]

    def rows(c):
        return pl.ds(base + c * _CHUNK, _CHUNK)

    def buf(k):
        return tile_bufs[k]

    def start_read(c):
        k = c % _NSLOT
        for i in range(_N_IN):
            @pl.when(preds[i])
            def _(i=i, k=k, c=c):
                pltpu.async_copy(xs[i].at[rows(c)], buf(k), rsems[k])

    def wait_read(c):
        k = c % _NSLOT
        pltpu.make_async_copy(xs[0].at[rows(c)], buf(k), rsems[k]).wait()

    def start_write(c):
        k = c % _NSLOT
        pltpu.async_copy(buf(k), out.at[rows(c)], wsems[k])

    def wait_write(c):
        k = c % _NSLOT
        pltpu.make_async_copy(buf(k), out.at[rows(c)], wsems[k]).wait()

    for c in range(min(_NSLOT, _NCH)):
        start_read(c)

    for c in range(_NCH):
        wait_read(c)
        start_write(c)
        prev = c - 1
        nxt = prev + _NSLOT
        if prev >= 0 and nxt < _NCH:
            wait_write(prev)
            start_read(nxt)
    for c in range(max(0, _NCH - _NSLOT), _NCH):
        wait_write(c)


def kernel(x0, x1, x2, x3, sel):
    sel_arr = jnp.full((16,), sel, dtype=jnp.int32)
    mesh = plsc.VectorSubcoreMesh(
        core_axis_name="c", subcore_axis_name="s",
        num_cores=_NC, num_subcores=_NS)
    mux = pl.kernel(
        _mux_body,
        out_type=jax.ShapeDtypeStruct((_B, _D), jnp.float32),
        mesh=mesh,
        compiler_params=pltpu.CompilerParams(needs_layout_passes=False),
        scratch_types=(
            [pltpu.VMEM((16,), jnp.int32)]
            + [pltpu.VMEM((_CHUNK, _D), jnp.float32) for _ in range(_NSLOT)]
            + [pltpu.SemaphoreType.DMA for _ in range(2 * _NSLOT)]
        ),
    )
    return mux(x0, x1, x2, x3, sel_arr)


# speedup vs baseline: 1.0219x; 1.0219x over previous
import jax
import jax.numpy as jnp
from jax import lax
from jax.experimental import pallas as pl
from jax.experimental.pallas import tpu as pltpu
from jax.experimental.pallas import tpu_sc as plsc

_B, _D = 8192, 2048
_N_IN = 4
_NC, _NS = 2, 16
_NW = _NC * _NS
_ROWS_W = _B // _NW

_CH_A, _NSLOT_A, _NCH_A = 16, 2, 10
_CH_B, _NSLOT_B, _NCH_B = 8, 3, 12
assert _CH_A * _NCH_A + _CH_B * _NCH_B == _ROWS_W


class _Ring:
    def __init__(self, xs, out, preds, bufs, rsems, wsems, row0, chunk, nch,
                 nslot):
        self.xs, self.out, self.preds = xs, out, preds
        self.bufs, self.rsems, self.wsems = bufs, rsems, wsems
        self.row0, self.chunk, self.nch, self.nslot = row0, chunk, nch, nslot

    def rows(self, c):
        return pl.ds(self.row0 + c * self.chunk, self.chunk)

    def start_read(self, c):
        k = c % self.nslot
        for i in range(_N_IN):
            @pl.when(self.preds[i])
            def _(i=i, k=k, c=c):
                pltpu.async_copy(self.xs[i].at[self.rows(c)], self.bufs[k],
                                 self.rsems[k])

    def wait_read(self, c):
        k = c % self.nslot
        pltpu.make_async_copy(self.xs[0].at[self.rows(c)], self.bufs[k],
                              self.rsems[k]).wait()

    def start_write(self, c):
        k = c % self.nslot
        pltpu.async_copy(self.bufs[k], self.out.at[self.rows(c)],
                         self.wsems[k])

    def wait_write(self, c):
        k = c % self.nslot
        pltpu.make_async_copy(self.bufs[k], self.out.at[self.rows(c)],
                              self.wsems[k]).wait()

    def groups(self):
        gs = []
        for c in range(min(self.nslot, self.nch)):
            gs.append([lambda c=c: self.start_read(c)])
        for c in range(self.nch):
            g = [lambda c=c: self.wait_read(c),
                 lambda c=c: self.start_write(c)]
            prev, nxt = c - 1, c - 1 + self.nslot
            if prev >= 0 and nxt < self.nch:
                g.append(lambda prev=prev: self.wait_write(prev))
                g.append(lambda nxt=nxt: self.start_read(nxt))
            gs.append(g)
        for c in range(max(0, self.nch - self.nslot), self.nch):
            gs.append([lambda c=c: self.wait_write(c)])
        return gs


def _mux_body(x0, x1, x2, x3, sel_hbm, out, sel_v, stage_sh, *rest):
    xs = (x0, x1, x2, x3)
    bufs_a = rest[:_NSLOT_A]
    sems = rest[_NSLOT_A:]
    rsems_a = sems[:_NSLOT_A]
    wsems_a = sems[_NSLOT_A:2 * _NSLOT_A]
    rsems_b = sems[2 * _NSLOT_A:2 * _NSLOT_A + _NSLOT_B]
    wsems_b = sems[2 * _NSLOT_A + _NSLOT_B:]

    sid = lax.axis_index("s")
    wid = sid * _NC + lax.axis_index("c")
    base = wid * _ROWS_W

    pltpu.sync_copy(sel_hbm, sel_v)
    selv = sel_v[...]
    preds = [jnp.any(selv == i) for i in range(_N_IN)]

    bufs_b = tuple(stage_sh.at[sid, k] for k in range(_NSLOT_B))
    ring_a = _Ring(xs, out, preds, bufs_a, rsems_a, wsems_a,
                   base, _CH_A, _NCH_A, _NSLOT_A)
    ring_b = _Ring(xs, out, preds, bufs_b, rsems_b, wsems_b,
                   base + _CH_A * _NCH_A, _CH_B, _NCH_B, _NSLOT_B)

    ga, gb = ring_a.groups(), ring_b.groups()
    n = max(len(ga), len(gb))
    for j in range(n):
        for gs in (ga, gb):
            if j < len(gs):
                for op in gs[j]:
                    op()


def kernel(x0, x1, x2, x3, sel):
    sel_arr = jnp.full((16,), sel, dtype=jnp.int32)
    mesh = plsc.VectorSubcoreMesh(
        core_axis_name="c", subcore_axis_name="s",
        num_cores=_NC, num_subcores=_NS)
    mux = pl.kernel(
        _mux_body,
        out_type=jax.ShapeDtypeStruct((_B, _D), jnp.float32),
        mesh=mesh,
        compiler_params=pltpu.CompilerParams(needs_layout_passes=False),
        scratch_types=(
            [pltpu.VMEM((16,), jnp.int32),
             pltpu.MemorySpace.VMEM_SHARED((_NS, _NSLOT_B, _CH_B, _D),
                                           jnp.float32)]
            + [pltpu.VMEM((_CH_A, _D), jnp.float32)
               for _ in range(_NSLOT_A)]
            + [pltpu.SemaphoreType.DMA
               for _ in range(2 * _NSLOT_A + 2 * _NSLOT_B)]
        ),
    )
    return mux(x0, x1, x2, x3, sel_arr)


# speedup vs baseline: 1.0454x; 1.0230x over previous
import jax
import jax.numpy as jnp
from jax import lax
from jax.experimental import pallas as pl
from jax.experimental.pallas import tpu as pltpu
from jax.experimental.pallas import tpu_sc as plsc

_B, _D = 8192, 2048
_N_IN = 4
_NC, _NS = 2, 16
_NW = _NC * _NS
_ROWS_W = _B // _NW
_CHUNK = 16
_NCH = _ROWS_W // _CHUNK
_NSLOT = 3


def _mux_body(x0, x1, x2, x3, sel_hbm, out, sel_v, stage_sh, *sems):
    xs = (x0, x1, x2, x3)
    rsems = sems[:_NSLOT]
    wsems = sems[_NSLOT:]

    sid = lax.axis_index("s")
    wid = sid * _NC + lax.axis_index("c")
    base = wid * _ROWS_W

    pltpu.sync_copy(sel_hbm, sel_v)
    selv = sel_v[...]
    preds = [jnp.any(selv == i) for i in range(_N_IN)]

    def rows(c):
        return pl.ds(base + c * _CHUNK, _CHUNK)

    def buf(k):
        return stage_sh.at[sid, k]

    def start_read(c):
        k = c % _NSLOT
        for i in range(_N_IN):
            @pl.when(preds[i])
            def _(i=i, k=k, c=c):
                pltpu.async_copy(xs[i].at[rows(c)], buf(k), rsems[k])

    def wait_read(c):
        k = c % _NSLOT
        pltpu.make_async_copy(xs[0].at[rows(c)], buf(k), rsems[k]).wait()

    def start_write(c):
        k = c % _NSLOT
        pltpu.async_copy(buf(k), out.at[rows(c)], wsems[k])

    def wait_write(c):
        k = c % _NSLOT
        pltpu.make_async_copy(buf(k), out.at[rows(c)], wsems[k]).wait()

    for c in range(min(_NSLOT, _NCH)):
        start_read(c)

    for c in range(_NCH):
        wait_read(c)
        start_write(c)
        prev = c - 1
        nxt = prev + _NSLOT
        if prev >= 0 and nxt < _NCH:
            wait_write(prev)
            start_read(nxt)
    for c in range(max(0, _NCH - _NSLOT), _NCH):
        wait_write(c)


def kernel(x0, x1, x2, x3, sel):
    sel_arr = jnp.full((16,), sel, dtype=jnp.int32)
    mesh = plsc.VectorSubcoreMesh(
        core_axis_name="c", subcore_axis_name="s",
        num_cores=_NC, num_subcores=_NS)
    mux = pl.kernel(
        _mux_body,
        out_type=jax.ShapeDtypeStruct((_B, _D), jnp.float32),
        mesh=mesh,
        compiler_params=pltpu.CompilerParams(needs_layout_passes=False),
        scratch_types=(
            [pltpu.VMEM((16,), jnp.int32),
             pltpu.MemorySpace.VMEM_SHARED((_NS, _NSLOT, _CHUNK, _D),
                                           jnp.float32)]
            + [pltpu.SemaphoreType.DMA for _ in range(2 * _NSLOT)]
        ),
    )
    return mux(x0, x1, x2, x3, sel_arr)
